# baseline (speedup 1.0000x reference)
import math

import jax
import jax.numpy as jnp
from jax import lax
from jax.experimental import pallas as pl
from jax.experimental.pallas import tpu as pltpu
from jax.experimental.pallas import tpu_sc as plsc

N = 10000
E = 320000
D = 128
NUM_BRDFS = 4
NUM_BOUNCES = 8
FSM_GAMMA = 1e-3
SPEED_OF_SOUND = 343.0
MEAN_FREE_PATH = 5.0
AIR_ABS = 1e-3

WSCALE = math.exp(-AIR_ABS * MEAN_FREE_PATH) / 64.0
DECAY = math.exp(math.log(FSM_GAMMA) * (MEAN_FREE_PATH / SPEED_OF_SOUND))

NC = 2
NS = 16
L = 16
DH = D // NC
B = 128
NPAD = 10240
RPT = NPAD // NS
RCH = RPT // B
EPAD = 327680
EPT = EPAD // NS
NB = EPT // B
NROWS2 = NC * NPAD


def _sc_body(x_hbm, row_hbm, col_hbm, attr_hbm, coef_hbm,
             out_hbm, rcw_hbm,
             racc, rcur,
             g0, g1, g2, g3, rb0, rb1, rb2, rb3, rb4, rb5, rb6, rb7,
             abuf, coef_v,
             sg0, sg1, sg2, sg3, ss0, ss1, ss2, ss3,
             sr0, sr1, sr2, sr3, sr4, sr5, sr6, sr7):
    gb = (g0, g1, g2, g3)
    rb = (rb0, rb1, rb2, rb3, rb4, rb5, rb6, rb7)
    sg = (sg0, sg1, sg2, sg3)
    ss = (ss0, ss1, ss2, ss3)
    sr = (sr0, sr1, sr2, sr3, sr4, sr5, sr6, sr7)
    c = lax.axis_index("c")
    s = lax.axis_index("s")
    rbase = c * NPAD + s * RPT
    lbase = s * RPT

    pltpu.sync_copy(coef_hbm, coef_v)
    cvec = coef_v[pl.ds(0, L)]
    c0 = cvec[0] * WSCALE
    c1 = cvec[1] * WSCALE
    c2 = cvec[2] * WSCALE
    c3 = cvec[3] * WSCALE

    def init_batch(b, _):
        pltpu.sync_copy(row_hbm.at[s, b], rb0.at[0])
        pltpu.sync_copy(col_hbm.at[s, b], rb0.at[1])
        for k in range(NUM_BRDFS):
            pltpu.sync_copy(attr_hbm.at[pl.ds(k * EPAD + s * EPT + b * B, B)],
                            abuf.at[k])
        for h in range(B // L):
            sl = pl.ds(h * L, L)
            w = (abuf[0, sl] * c0 + abuf[1, sl] * c1
                 + abuf[2, sl] * c2 + abuf[3, sl] * c3)
            rb0[2, sl] = plsc.bitcast(w, jnp.int32)
        pltpu.sync_copy(rb0, rcw_hbm.at[s, b])
        return 0

    lax.fori_loop(0, NB, init_batch, 0, unroll=False)

    def init_rows(j, _):
        pltpu.sync_copy(x_hbm.at[pl.ds(rbase + j * B, B)], g0)
        pltpu.sync_copy(g0, rcur.at[pl.ds(lbase + j * B, B)])
        pltpu.sync_copy(g0, out_hbm.at[pl.ds(rbase + j * B, B)])

        def zz(e, _):
            for f in range(DH // L):
                g0[e, pl.ds(f * L, L)] = jnp.zeros((L,), jnp.float32)
            return 0

        lax.fori_loop(0, B, zz, 0, unroll=False)
        pltpu.sync_copy(g0, racc.at[pl.ds(lbase + j * B, B)])
        return 0

    lax.fori_loop(0, RCH, init_rows, 0, unroll=False)
    plsc.subcore_barrier()

    def bounce(t, _):
        for m in range(4):
            pltpu.async_copy(rcw_hbm.at[s, m], rb[m], sr[m])
        pltpu.make_async_copy(rcw_hbm.at[s, 0], rb[0], sr[0]).wait()
        pltpu.async_copy(rcur.at[rb[0].at[0]], gb[0], sg[0])
        pltpu.make_async_copy(rcw_hbm.at[s, 0], rb[1], sr[1]).wait()
        pltpu.async_copy(rcur.at[rb[1].at[0]], gb[1], sg[1])
        pltpu.async_copy(racc.at[pl.ds(0, B)], gb[2], ss[2])
        pltpu.async_copy(racc.at[pl.ds(0, B)], gb[3], ss[3])

        def oct_body(ho, _):
            for u in range(8):
                j = ho * 8 + u
                p = u % 4
                q = (u + 2) % 4
                m = u
                m2 = (u + 2) % 8
                m4 = (u + 4) % 8
                buf = gb[p]
                pltpu.make_async_copy(rcur.at[rb[m].at[0]], buf,
                                      sg[p]).wait()

                def mult_h(h, _):
                    wv = plsc.bitcast(rb[m][2, pl.ds(h * L, L)], jnp.float32)
                    for jj in range(L):
                        wj = jnp.full((L,), wv[jj], jnp.float32)
                        e = h * L + jj
                        for f in range(DH // L):
                            sl = pl.ds(f * L, L)
                            buf[e, sl] = buf[e, sl] * wj
                    return 0

                lax.fori_loop(0, B // L, mult_h, 0, unroll=False)
                pltpu.async_copy(buf, racc.at[rb[m].at[1]], ss[p], add=True)
                pltpu.make_async_copy(gb[q], racc.at[rb[m].at[1]],
                                      ss[q]).wait()
                pltpu.make_async_copy(rcw_hbm.at[s, 0], rb[m2], sr[m2]).wait()
                pltpu.async_copy(rcur.at[rb[m2].at[0]], gb[q], sg[q])
                jn = jnp.minimum(j + 4, NB - 1)
                pltpu.async_copy(rcw_hbm.at[s, jn], rb[m4], sr[m4])
            return 0

        lax.fori_loop(0, NB // 8, oct_body, 0, unroll=False)
        pltpu.make_async_copy(rcur.at[rb[0].at[0]], gb[0], sg[0]).wait()
        pltpu.make_async_copy(rcur.at[rb[0].at[0]], gb[1], sg[1]).wait()
        pltpu.make_async_copy(gb[2], racc.at[rb[0].at[1]], ss[2]).wait()
        pltpu.make_async_copy(gb[3], racc.at[rb[0].at[1]], ss[3]).wait()
        pltpu.make_async_copy(rcw_hbm.at[s, 0], rb[2], sr[2]).wait()
        pltpu.make_async_copy(rcw_hbm.at[s, 0], rb[3], sr[3]).wait()
        plsc.subcore_barrier()

        def chunk_body(j, _):
            pltpu.sync_copy(racc.at[pl.ds(lbase + j * B, B)], g0)
            pltpu.sync_copy(out_hbm.at[pl.ds(rbase + j * B, B)], g1)

            def row_body(e, _):
                for f in range(DH // L):
                    sl = pl.ds(f * L, L)
                    v = g0[e, sl] * DECAY
                    g0[e, sl] = v
                    g1[e, sl] = g1[e, sl] + v
                return 0

            lax.fori_loop(0, B, row_body, 0, unroll=False)
            pltpu.sync_copy(g0, rcur.at[pl.ds(lbase + j * B, B)])
            pltpu.sync_copy(g1, out_hbm.at[pl.ds(rbase + j * B, B)])

            def zz(e, _):
                for f in range(DH // L):
                    g0[e, pl.ds(f * L, L)] = jnp.zeros((L,), jnp.float32)
                return 0

            lax.fori_loop(0, B, zz, 0, unroll=False)
            pltpu.sync_copy(g0, racc.at[pl.ds(lbase + j * B, B)])
            return 0

        lax.fori_loop(0, RCH, chunk_body, 0, unroll=False)
        plsc.subcore_barrier()
        return 0

    lax.fori_loop(0, NUM_BOUNCES, bounce, 0, unroll=False)


def kernel(x, edge_index, edge_attr, brdf_coeffs):
    x2 = x.reshape(N, NC, DH).transpose(1, 0, 2)
    x2 = jnp.pad(x2, ((0, 0), (0, NPAD - N), (0, 0))).reshape(NROWS2, DH)

    row = jnp.pad(edge_index[0].astype(jnp.int32), (0, EPAD - E))
    col = jnp.pad(edge_index[1].astype(jnp.int32), (0, EPAD - E))
    row2 = row.reshape(NS, NB, B)
    col2 = col.reshape(NS, NB, B)
    attr2 = jnp.pad(edge_attr.astype(jnp.float32).T,
                    ((0, 0), (0, EPAD - E))).reshape(NUM_BRDFS * EPAD)
    coef = jnp.pad(brdf_coeffs.astype(jnp.float32), (0, L - NUM_BRDFS))

    mesh = plsc.VectorSubcoreMesh(core_axis_name="c", subcore_axis_name="s",
                                  num_cores=NC, num_subcores=NS)
    f32 = jnp.float32
    i32 = jnp.int32
    run = pl.kernel(
        _sc_body,
        out_type=(jax.ShapeDtypeStruct((NROWS2, DH), f32),
                  jax.ShapeDtypeStruct((NS, NB, 3, B), i32)),
        mesh=mesh,
        scratch_types=[
            pltpu.VMEM_SHARED((NPAD, DH), f32),
            pltpu.VMEM_SHARED((NPAD, DH), f32),
            pltpu.VMEM((B, DH), f32),
            pltpu.VMEM((B, DH), f32),
            pltpu.VMEM((B, DH), f32),
            pltpu.VMEM((B, DH), f32),
            pltpu.VMEM((3, B), i32),
            pltpu.VMEM((3, B), i32),
            pltpu.VMEM((3, B), i32),
            pltpu.VMEM((3, B), i32),
            pltpu.VMEM((3, B), i32),
            pltpu.VMEM((3, B), i32),
            pltpu.VMEM((3, B), i32),
            pltpu.VMEM((3, B), i32),
            pltpu.VMEM((NUM_BRDFS, B), f32),
            pltpu.VMEM((L,), f32),
            pltpu.SemaphoreType.DMA,
            pltpu.SemaphoreType.DMA,
            pltpu.SemaphoreType.DMA,
            pltpu.SemaphoreType.DMA,
            pltpu.SemaphoreType.DMA,
            pltpu.SemaphoreType.DMA,
            pltpu.SemaphoreType.DMA,
            pltpu.SemaphoreType.DMA,
            pltpu.SemaphoreType.DMA,
            pltpu.SemaphoreType.DMA,
            pltpu.SemaphoreType.DMA,
            pltpu.SemaphoreType.DMA,
            pltpu.SemaphoreType.DMA,
            pltpu.SemaphoreType.DMA,
            pltpu.SemaphoreType.DMA,
            pltpu.SemaphoreType.DMA,
        ],
        compiler_params=pltpu.CompilerParams(use_tc_tiling_on_sc=False,
                                             needs_layout_passes=False),
    )
    out2, _ = run(x2, row2, col2, attr2, coef)
    out = out2.reshape(NC, NPAD, DH)[:, :N]
    return out.transpose(1, 0, 2).reshape(N, D)
